# Initial kernel scaffold; baseline (speedup 1.0000x reference)
import jax
import jax.numpy as jnp
from jax.experimental import pallas as pl


def kernel(x):
    raise NotImplementedError("write your pallas kernel here")


# speedup vs baseline: 8.4996x; 8.4996x over previous
import numpy as np
import jax
import jax.numpy as jnp
from jax.experimental import pallas as pl
from jax.experimental.pallas import tpu as pltpu

_NUM_UPSAMPLE = 2


def _stage_key_consts():
    consts = []
    root = jax.random.key(42)
    for i in range(_NUM_UPSAMPLE):
        k = jax.random.fold_in(root, i)
        k1, k2 = jax.random.split(k)
        k2b = jax.random.split(k2)[1]
        ud = np.asarray(jax.random.key_data(k1)).astype(np.uint32)
        md = np.asarray(jax.random.key_data(k2b)).astype(np.uint32)
        consts.append((int(ud[0]), int(ud[1]), int(md[0]), int(md[1])))
    return consts


_KEYS = _stage_key_consts()


def _rotl(x, r):
    return (x << np.uint32(r)) | (x >> np.uint32(32 - r))


def _tf_bits(k0, k1, idx):
    ks0 = np.uint32(k0)
    ks1 = np.uint32(k1)
    ks2 = np.uint32(ks0 ^ ks1 ^ np.uint32(0x1BD11BDA))
    ks = (ks0, ks1, ks2)
    rots = ((13, 15, 26, 6), (17, 29, 16, 24))
    x0 = jnp.full(idx.shape, ks0, dtype=jnp.uint32)
    x1 = idx + ks1
    for i in range(5):
        for r in rots[i % 2]:
            x0 = x0 + x1
            x1 = _rotl(x1, r)
            x1 = x0 ^ x1
        x0 = x0 + ks[(i + 1) % 3]
        x1 = x1 + np.uint32((int(ks[(i + 2) % 3]) + i + 1) & 0xFFFFFFFF)
    return x0 ^ x1


def _u01(bits):
    fb = (bits >> np.uint32(9)) | np.uint32(0x3F800000)
    return jax.lax.bitcast_convert_type(fb, jnp.float32) - jnp.float32(1.0)


def _pdf(u):
    return jnp.exp(jnp.float32(-0.5) * u * u)


def _body(x_ref, ecat_ref, out_ref, *, ht, h, w, b_dim):
    b = jax.lax.convert_element_type(pl.program_id(0), jnp.uint32)
    t = jax.lax.convert_element_type(pl.program_id(1), jnp.uint32)
    h0 = t * np.uint32(ht)

    uk0, uk1, mk0, mk1 = _KEYS[0]
    uk0b, uk1b, mk0b, mk1b = _KEYS[1]

    xb = x_ref[0, 0]

    ri = jax.lax.broadcasted_iota(jnp.uint32, (ht, w), 0)
    ci = jax.lax.broadcasted_iota(jnp.uint32, (ht, w), 1)
    i1 = (b * np.uint32(h * w * 4)
          + (h0 + ri) * np.uint32(w * 4) + ci * np.uint32(4))
    pdfs = []
    vals = []
    for k in range(4):
        ik = i1 + np.uint32(k)
        pk = _pdf(_u01(_tf_bits(uk0, uk1, ik)))
        mk = _tf_bits(mk0, mk1, ik) & np.uint32(1)
        pdfs.append(pk)
        vals.append(jnp.where(mk == np.uint32(1), jnp.float32(0.0), pk))
    s1 = (vals[0] + vals[1]) + (vals[2] + vals[3])
    ok1 = s1 > 0
    vals = [jnp.where(ok1, v, p) for v, p in zip(vals, pdfs)]
    s1 = (vals[0] + vals[1]) + (vals[2] + vals[3])
    inv1 = jnp.float32(1.0) / s1
    p = [xb * (v * inv1) for v in vals]

    rn = 4 * ht
    rri = jax.lax.broadcasted_iota(jnp.uint32, (rn, ht), 0)
    hhi = jax.lax.broadcasted_iota(jnp.uint32, (rn, ht), 1)
    sel_h = (rri >> np.uint32(2)) == hhi
    l0 = jnp.where(sel_h & (((rri >> np.uint32(1)) & np.uint32(1)) == 0),
                   jnp.float32(1.0), jnp.float32(0.0))
    l1 = jnp.where(sel_h & (((rri >> np.uint32(1)) & np.uint32(1)) == 1),
                   jnp.float32(1.0), jnp.float32(0.0))
    dot = lambda a, c: jax.lax.dot(a, c, preferred_element_type=jnp.float32)
    a0 = dot(l0, p[0]) + dot(l1, p[2])
    a1 = dot(l0, p[1]) + dot(l1, p[3])
    acat = jnp.concatenate([a0, a1], axis=1)
    y = dot(acat, ecat_ref[...])

    r2 = jax.lax.broadcasted_iota(jnp.uint32, (rn, 4 * w), 0)
    c2 = jax.lax.broadcasted_iota(jnp.uint32, (rn, 4 * w), 1)
    h2 = ((h0 + (r2 >> np.uint32(2))) * np.uint32(2)
          + ((r2 >> np.uint32(1)) & np.uint32(1)))
    w2 = (c2 >> np.uint32(2)) * np.uint32(2) + ((c2 >> np.uint32(1)) & np.uint32(1))
    k2 = (r2 & np.uint32(1)) * np.uint32(2) + (c2 & np.uint32(1))
    i2 = (b * np.uint32(4 * h * 4 * w * 4 // 4)
          + h2 * np.uint32(2 * w * 4) + w2 * np.uint32(4) + k2)
    pdf2 = _pdf(_u01(_tf_bits(uk0b, uk1b, i2)))
    m2 = _tf_bits(mk0b, mk1b, i2) & np.uint32(1)
    v2 = jnp.where(m2 == np.uint32(1), jnp.float32(0.0), pdf2)

    col_even = (c2 & np.uint32(1)) == 0
    rpi = jax.lax.broadcasted_iota(jnp.uint32, (rn, rn), 0)
    rpj = jax.lax.broadcasted_iota(jnp.uint32, (rn, rn), 1)
    rowpair = jnp.where((rpi >> np.uint32(1)) == (rpj >> np.uint32(1)),
                        jnp.float32(1.0), jnp.float32(0.0))

    def quadsum(v):
        cp = v + jnp.where(col_even, jnp.roll(v, -1, axis=1),
                           jnp.roll(v, 1, axis=1))
        return dot(rowpair, cp)

    s2 = quadsum(v2)
    v2 = jnp.where(s2 > 0, v2, pdf2)
    s2 = quadsum(v2)
    out_ref[0, 0] = y * (v2 * (jnp.float32(1.0) / s2))


def kernel(x):
    b_dim, c_dim, h, w = x.shape
    assert c_dim == 1
    ht = min(8, h)
    grid = (b_dim, h // ht)

    wi = np.arange(2 * w)[:, None]
    c = np.arange(4 * w)[None, :]
    ecat = (((c >> 2) == (wi % w)) & (((c >> 1) & 1) == (wi // w))).astype(np.float32)

    out = pl.pallas_call(
        lambda xr, er, orf: _body(xr, er, orf, ht=ht, h=h, w=w, b_dim=b_dim),
        grid=grid,
        in_specs=[
            pl.BlockSpec((1, 1, ht, w), lambda i, j: (i, 0, j, 0)),
            pl.BlockSpec((2 * w, 4 * w), lambda i, j: (0, 0)),
        ],
        out_specs=pl.BlockSpec((1, 1, 4 * ht, 4 * w), lambda i, j: (i, 0, j, 0)),
        out_shape=jax.ShapeDtypeStruct((b_dim, 1, 4 * h, 4 * w), jnp.float32),
        compiler_params=pltpu.CompilerParams(
            dimension_semantics=("parallel", "parallel"),
        ),
    )(x, jnp.asarray(ecat))
    return out


# speedup vs baseline: 8.7198x; 1.0259x over previous
import numpy as np
import jax
import jax.numpy as jnp
from jax.experimental import pallas as pl
from jax.experimental.pallas import tpu as pltpu

_NUM_UPSAMPLE = 2


def _tf_scalar(key, x0, x1):
    M = 0xFFFFFFFF
    k0, k1 = int(key[0]), int(key[1])
    ks = (k0, k1, k0 ^ k1 ^ 0x1BD11BDA)
    rots = ((13, 15, 26, 6), (17, 29, 16, 24))
    x0 = (x0 + ks[0]) & M
    x1 = (x1 + ks[1]) & M
    for i in range(5):
        for r in rots[i % 2]:
            x0 = (x0 + x1) & M
            x1 = ((x1 << r) | (x1 >> (32 - r))) & M
            x1 = x0 ^ x1
        x0 = (x0 + ks[(i + 1) % 3]) & M
        x1 = (x1 + ks[(i + 2) % 3] + i + 1) & M
    return x0, x1


def _stage_key_consts():
    consts = []
    root = (0, 42)
    for i in range(_NUM_UPSAMPLE):
        k = _tf_scalar(root, 0, i)
        k1 = _tf_scalar(k, 0, 0)
        k2 = _tf_scalar(k, 0, 1)
        k2b = _tf_scalar(k2, 0, 1)
        consts.append((k1[0], k1[1], k2b[0], k2b[1]))
    return consts


_KEYS = _stage_key_consts()


def _rotl(x, r):
    return (x << np.uint32(r)) | (x >> np.uint32(32 - r))


def _tf_bits(k0, k1, idx):
    ks0 = np.uint32(k0)
    ks1 = np.uint32(k1)
    ks2 = np.uint32(ks0 ^ ks1 ^ np.uint32(0x1BD11BDA))
    ks = (ks0, ks1, ks2)
    rots = ((13, 15, 26, 6), (17, 29, 16, 24))
    x0 = jnp.full(idx.shape, ks0, dtype=jnp.uint32)
    x1 = idx + ks1
    for i in range(5):
        for r in rots[i % 2]:
            x0 = x0 + x1
            x1 = _rotl(x1, r)
            x1 = x0 ^ x1
        x0 = x0 + ks[(i + 1) % 3]
        x1 = x1 + np.uint32((int(ks[(i + 2) % 3]) + i + 1) & 0xFFFFFFFF)
    return x0 ^ x1


def _u01(bits):
    fb = (bits >> np.uint32(9)) | np.uint32(0x3F800000)
    return jax.lax.bitcast_convert_type(fb, jnp.float32) - jnp.float32(1.0)


def _pdf(u):
    return jnp.exp(jnp.float32(-0.5) * u * u)


def _body(x_ref, ecat_ref, g1_ref, g2_ref, lcat_ref, rp_ref, out_ref,
          *, ht, h, w):
    b = jax.lax.convert_element_type(pl.program_id(0), jnp.uint32)
    t = jax.lax.convert_element_type(pl.program_id(1), jnp.uint32)
    h0 = t * np.uint32(ht)

    uk0, uk1, mk0, mk1 = _KEYS[0]
    uk0b, uk1b, mk0b, mk1b = _KEYS[1]

    xb = x_ref[0, 0]
    rn = 4 * ht

    base1 = b * np.uint32(h * w * 4) + h0 * np.uint32(w * 4)
    g1 = g1_ref[...]
    pdfs = []
    vals = []
    for k in range(4):
        ik = g1 + (base1 + np.uint32(k))
        pk = _pdf(_u01(_tf_bits(uk0, uk1, ik)))
        mk = _tf_bits(mk0, mk1, ik) & np.uint32(1)
        pdfs.append(pk)
        vals.append(jnp.where(mk == np.uint32(1), jnp.float32(0.0), pk))
    s1 = (vals[0] + vals[1]) + (vals[2] + vals[3])
    ok1 = s1 > 0
    vals = [jnp.where(ok1, v, p) for v, p in zip(vals, pdfs)]
    s1 = (vals[0] + vals[1]) + (vals[2] + vals[3])
    inv1 = jnp.float32(1.0) / s1
    p = [xb * (v * inv1) for v in vals]

    dot = lambda a, c: jax.lax.dot(a, c, preferred_element_type=jnp.float32)
    lcat = lcat_ref[...]
    pc0 = jnp.concatenate([p[0], p[2]], axis=0)
    pc1 = jnp.concatenate([p[1], p[3]], axis=0)
    a0 = dot(lcat, pc0)
    a1 = dot(lcat, pc1)
    acat = jnp.concatenate([a0, a1], axis=1).astype(jnp.bfloat16)
    y = dot(acat, ecat_ref[...])

    i2 = g2_ref[...] + (b * np.uint32(16 * h * w) + h0 * np.uint32(16 * w))
    pdf2 = _pdf(_u01(_tf_bits(uk0b, uk1b, i2)))
    m2 = _tf_bits(mk0b, mk1b, i2) & np.uint32(1)
    v2 = jnp.where(m2 == np.uint32(1), jnp.float32(0.0), pdf2)

    c2 = jax.lax.broadcasted_iota(jnp.uint32, (rn, 4 * w), 1)
    col_even = (c2 & np.uint32(1)) == 0
    rowpair = rp_ref[...]

    def quadsum(v):
        cp = v + jnp.where(col_even, jnp.roll(v, -1, axis=1),
                           jnp.roll(v, 1, axis=1))
        return dot(rowpair, cp)

    s2 = quadsum(v2)
    v2 = jnp.where(s2 > 0, v2, pdf2)
    s2 = quadsum(v2)
    out_ref[0, 0] = y * (v2 * (jnp.float32(1.0) / s2))


def kernel(x):
    b_dim, c_dim, h, w = x.shape
    assert c_dim == 1
    ht = min(8, h)
    rn = 4 * ht
    grid = (b_dim, h // ht)

    wi = np.arange(2 * w)[:, None]
    c = np.arange(4 * w)[None, :]
    ecat = (((c >> 2) == (wi % w)) & (((c >> 1) & 1) == (wi // w)))
    ecat = jnp.asarray(ecat.astype(np.float32), dtype=jnp.bfloat16)

    rr = np.arange(rn)[:, None]
    j = np.arange(2 * ht)[None, :]
    lcat = (((rr >> 2) == (j % ht)) & (((rr >> 1) & 1) == (j // ht)))
    lcat = jnp.asarray(lcat.astype(np.float32))

    ri = np.arange(rn)[:, None]
    rj = np.arange(rn)[None, :]
    rowpair = jnp.asarray(((ri >> 1) == (rj >> 1)).astype(np.float32))

    hh = np.arange(ht)[:, None].astype(np.uint32)
    ww = np.arange(w)[None, :].astype(np.uint32)
    g1 = jnp.asarray(hh * np.uint32(w * 4) + ww * np.uint32(4))
    rr2 = np.arange(rn)[:, None]
    cc2 = np.arange(4 * w)[None, :]
    h2pat = 2 * (rr2 >> 2) + ((rr2 >> 1) & 1)
    w2pat = 2 * (cc2 >> 2) + ((cc2 >> 1) & 1)
    k2pat = 2 * (rr2 & 1) + (cc2 & 1)
    g2 = jnp.asarray((h2pat * (8 * w) + w2pat * 4 + k2pat).astype(np.uint32))

    const_spec = lambda shp: pl.BlockSpec(shp, lambda i, jj: (0, 0))
    out = pl.pallas_call(
        lambda xr, er, g1r, g2r, lr, rpr, orf: _body(
            xr, er, g1r, g2r, lr, rpr, orf, ht=ht, h=h, w=w),
        grid=grid,
        in_specs=[
            pl.BlockSpec((1, 1, ht, w), lambda i, jj: (i, 0, jj, 0)),
            const_spec((2 * w, 4 * w)),
            const_spec((ht, w)),
            const_spec((rn, 4 * w)),
            const_spec((rn, 2 * ht)),
            const_spec((rn, rn)),
        ],
        out_specs=pl.BlockSpec((1, 1, rn, 4 * w), lambda i, jj: (i, 0, jj, 0)),
        out_shape=jax.ShapeDtypeStruct((b_dim, 1, 4 * h, 4 * w), jnp.float32),
        compiler_params=pltpu.CompilerParams(
            dimension_semantics=("parallel", "parallel"),
        ),
    )(x, ecat, g1, g2, lcat, rowpair)
    return out


# speedup vs baseline: 14.1804x; 1.6262x over previous
import numpy as np
import jax
import jax.numpy as jnp
from jax.experimental import pallas as pl
from jax.experimental.pallas import tpu as pltpu

_NUM_UPSAMPLE = 2


def _tf_scalar(key, x0, x1):
    M = 0xFFFFFFFF
    k0, k1 = int(key[0]), int(key[1])
    ks = (k0, k1, k0 ^ k1 ^ 0x1BD11BDA)
    rots = ((13, 15, 26, 6), (17, 29, 16, 24))
    x0 = (x0 + ks[0]) & M
    x1 = (x1 + ks[1]) & M
    for i in range(5):
        for r in rots[i % 2]:
            x0 = (x0 + x1) & M
            x1 = ((x1 << r) | (x1 >> (32 - r))) & M
            x1 = x0 ^ x1
        x0 = (x0 + ks[(i + 1) % 3]) & M
        x1 = (x1 + ks[(i + 2) % 3] + i + 1) & M
    return x0, x1


def _stage_key_consts():
    consts = []
    root = (0, 42)
    for i in range(_NUM_UPSAMPLE):
        k = _tf_scalar(root, 0, i)
        k1 = _tf_scalar(k, 0, 0)
        k2 = _tf_scalar(k, 0, 1)
        k2b = _tf_scalar(k2, 0, 1)
        consts.append((k1[0], k1[1], k2b[0], k2b[1]))
    return consts


_KEYS = _stage_key_consts()


def _rotl(x, r):
    return (x << np.uint32(r)) | (x >> np.uint32(32 - r))


def _tf_bits(k0, k1, idx):
    ks0 = np.uint32(k0)
    ks1 = np.uint32(k1)
    ks2 = np.uint32(ks0 ^ ks1 ^ np.uint32(0x1BD11BDA))
    ks = (ks0, ks1, ks2)
    rots = ((13, 15, 26, 6), (17, 29, 16, 24))
    x0 = jnp.full(idx.shape, ks0, dtype=jnp.uint32)
    x1 = idx + ks1
    for i in range(5):
        for r in rots[i % 2]:
            x0 = x0 + x1
            x1 = _rotl(x1, r)
            x1 = x0 ^ x1
        x0 = x0 + ks[(i + 1) % 3]
        x1 = x1 + np.uint32((int(ks[(i + 2) % 3]) + i + 1) & 0xFFFFFFFF)
    return x0 ^ x1


def _u01(bits):
    fb = (bits >> np.uint32(9)) | np.uint32(0x3F800000)
    return jax.lax.bitcast_convert_type(fb, jnp.float32) - jnp.float32(1.0)


def _pdf(u):
    return jnp.exp(jnp.float32(-0.5) * u * u)


def _body(boff_ref, x_ref, ecat_ref, g1_ref, g2_ref, lcat_ref, rp_ref,
          out_ref, *, ht, h, w):
    b = jax.lax.convert_element_type(pl.program_id(0), jnp.uint32) + boff_ref[0]
    t = jax.lax.convert_element_type(pl.program_id(1), jnp.uint32)
    h0 = t * np.uint32(ht)

    uk0, uk1, mk0, mk1 = _KEYS[0]
    uk0b, uk1b, mk0b, mk1b = _KEYS[1]

    xb = x_ref[0, 0]
    rn = 4 * ht

    base1 = b * np.uint32(h * w * 4) + h0 * np.uint32(w * 4)
    g1 = g1_ref[...]
    pdfs = []
    vals = []
    for k in range(4):
        ik = g1 + (base1 + np.uint32(k))
        pk = _pdf(_u01(_tf_bits(uk0, uk1, ik)))
        mk = _tf_bits(mk0, mk1, ik) & np.uint32(1)
        pdfs.append(pk)
        vals.append(jnp.where(mk == np.uint32(1), jnp.float32(0.0), pk))
    s1 = (vals[0] + vals[1]) + (vals[2] + vals[3])
    ok1 = s1 > 0
    vals = [jnp.where(ok1, v, p) for v, p in zip(vals, pdfs)]
    s1 = (vals[0] + vals[1]) + (vals[2] + vals[3])
    inv1 = jnp.float32(1.0) / s1
    p = [xb * (v * inv1) for v in vals]

    dot = lambda a, c: jax.lax.dot(a, c, preferred_element_type=jnp.float32)
    lcat = lcat_ref[...]
    pc0 = jnp.concatenate([p[0], p[2]], axis=0)
    pc1 = jnp.concatenate([p[1], p[3]], axis=0)
    a0 = dot(lcat, pc0)
    a1 = dot(lcat, pc1)
    acat = jnp.concatenate([a0, a1], axis=1).astype(jnp.bfloat16)
    y = dot(acat, ecat_ref[...])

    i2 = g2_ref[...] + (b * np.uint32(16 * h * w) + h0 * np.uint32(16 * w))
    pdf2 = _pdf(_u01(_tf_bits(uk0b, uk1b, i2)))
    m2 = _tf_bits(mk0b, mk1b, i2) & np.uint32(1)
    v2 = jnp.where(m2 == np.uint32(1), jnp.float32(0.0), pdf2)

    c2 = jax.lax.broadcasted_iota(jnp.uint32, (rn, 4 * w), 1)
    col_even = (c2 & np.uint32(1)) == 0
    rowpair = rp_ref[...]

    def quadsum(v):
        cp = v + jnp.where(col_even, jnp.roll(v, -1, axis=1),
                           jnp.roll(v, 1, axis=1))
        return dot(rowpair, cp)

    s2 = quadsum(v2)
    v2 = jnp.where(s2 > 0, v2, pdf2)
    s2 = quadsum(v2)
    out_ref[0, 0] = y * (v2 * (jnp.float32(1.0) / s2))


def _run(x, boff):
    b_dim, c_dim, h, w = x.shape
    assert c_dim == 1
    ht = min(8, h)
    rn = 4 * ht
    grid = (b_dim, h // ht)

    wi = np.arange(2 * w)[:, None]
    c = np.arange(4 * w)[None, :]
    ecat = (((c >> 2) == (wi % w)) & (((c >> 1) & 1) == (wi // w)))
    ecat = jnp.asarray(ecat.astype(np.float32), dtype=jnp.bfloat16)

    rr = np.arange(rn)[:, None]
    j = np.arange(2 * ht)[None, :]
    lcat = (((rr >> 2) == (j % ht)) & (((rr >> 1) & 1) == (j // ht)))
    lcat = jnp.asarray(lcat.astype(np.float32))

    ri = np.arange(rn)[:, None]
    rj = np.arange(rn)[None, :]
    rowpair = jnp.asarray(((ri >> 1) == (rj >> 1)).astype(np.float32))

    hh = np.arange(ht)[:, None].astype(np.uint32)
    ww = np.arange(w)[None, :].astype(np.uint32)
    g1 = jnp.asarray(hh * np.uint32(w * 4) + ww * np.uint32(4))
    rr2 = np.arange(rn)[:, None]
    cc2 = np.arange(4 * w)[None, :]
    h2pat = 2 * (rr2 >> 2) + ((rr2 >> 1) & 1)
    w2pat = 2 * (cc2 >> 2) + ((cc2 >> 1) & 1)
    k2pat = 2 * (rr2 & 1) + (cc2 & 1)
    g2 = jnp.asarray((h2pat * (8 * w) + w2pat * 4 + k2pat).astype(np.uint32))

    const_spec = lambda shp: pl.BlockSpec(shp, lambda i, jj: (0, 0))
    out = pl.pallas_call(
        lambda bo, xr, er, g1r, g2r, lr, rpr, orf: _body(
            bo, xr, er, g1r, g2r, lr, rpr, orf, ht=ht, h=h, w=w),
        grid=grid,
        in_specs=[
            pl.BlockSpec(memory_space=pltpu.SMEM),
            pl.BlockSpec((1, 1, ht, w), lambda i, jj: (i, 0, jj, 0)),
            const_spec((2 * w, 4 * w)),
            const_spec((ht, w)),
            const_spec((rn, 4 * w)),
            const_spec((rn, 2 * ht)),
            const_spec((rn, rn)),
        ],
        out_specs=pl.BlockSpec((1, 1, rn, 4 * w), lambda i, jj: (i, 0, jj, 0)),
        out_shape=jax.ShapeDtypeStruct((b_dim, 1, 4 * h, 4 * w), jnp.float32),
        compiler_params=pltpu.CompilerParams(
            dimension_semantics=("parallel", "parallel"),
        ),
    )(boff, x, ecat, g1, g2, lcat, rowpair)
    return out


def kernel(x):
    b_dim = x.shape[0]
    n_dev = jax.device_count()
    if n_dev >= 2 and b_dim % 2 == 0:
        mesh = jax.make_mesh((2,), ("d",))
        spec = jax.sharding.PartitionSpec("d")

        def shard_fn(xs):
            boff = (jax.lax.axis_index("d") * (b_dim // 2)).astype(
                jnp.uint32).reshape(1)
            return _run(xs, boff)

        xs = jax.reshard(x, jax.sharding.NamedSharding(mesh, spec))
        return jax.shard_map(shard_fn, mesh=mesh, in_specs=spec,
                             out_specs=spec, check_vma=False)(xs)
    return _run(x, jnp.zeros((1,), jnp.uint32))
